# baseline (speedup 1.0000x reference)
import functools

import jax
import jax.numpy as jnp
from jax import lax
from jax.experimental import pallas as pl
from jax.experimental.pallas import tpu as pltpu
from jax.experimental.pallas import tpu_sc as plsc

M = 16384
H = 2048
D = 4096
NCORE = 2
NSUB = 16
BMM = 256
CH = 16
CH2 = 8
NBUF = 3

_mesh = plsc.VectorSubcoreMesh(core_axis_name="c", subcore_axis_name="s")


def _make_sparse_stages(row0, nrows):
    half = nrows // NCORE
    chunk = half // NSUB
    nvec = chunk // 16
    regblk = half // BMM + 1
    reg = regblk * BMM
    mc = NCORE * reg

    @functools.partial(
        pl.kernel,
        out_type=(
            jax.ShapeDtypeStruct((mc, H), jnp.float32),
            jax.ShapeDtypeStruct((nrows,), jnp.int32),
            jax.ShapeDtypeStruct((NCORE * 16,), jnp.int32),
        ),
        mesh=_mesh,
        scratch_types=[
            pltpu.VMEM((half,), jnp.int32),
            pltpu.VMEM((chunk,), jnp.int32),
            pltpu.VMEM((chunk,), jnp.int32),
            pltpu.VMEM((16,), jnp.int32),
            pltpu.VMEM((CH, H), jnp.float32),
            pltpu.VMEM((CH, H), jnp.float32),
            pltpu.VMEM((CH, H), jnp.float32),
            pltpu.SemaphoreType.DMA,
            pltpu.SemaphoreType.DMA,
            pltpu.SemaphoreType.DMA,
            pltpu.SemaphoreType.DMA,
            pltpu.SemaphoreType.DMA,
            pltpu.SemaphoreType.DMA,
        ],
        compiler_params=pltpu.CompilerParams(needs_layout_passes=False),
    )
    def sc_compact(mask_hbm, x_hbm, cx_hbm, src_hbm, cnt_hbm,
                   mask_v, idx_v, src_v, stage_v, rb0, rb1, rb2,
                   g0, g1, g2, w0, w1, w2):
        rbs = (rb0, rb1, rb2)
        gsems = (g0, g1, g2)
        wsems = (w0, w1, w2)
        c = lax.axis_index("c")
        s = lax.axis_index("s")
        base = c * half + s * chunk

        pltpu.sync_copy(mask_hbm.at[pl.ds(row0 + c * half, half)], mask_v)

        off = jnp.int32(0)
        total = jnp.int32(0)
        cnt_s = jnp.int32(0)
        for t in range(NSUB):
            def _vsum(k, a, t=t):
                return a + mask_v[pl.ds(t * chunk + k * 16, 16)]
            v_t = jnp.sum(lax.fori_loop(0, nvec, _vsum,
                                        jnp.zeros((16,), jnp.int32)))
            p_t = ((v_t + CH - 1) // CH) * CH
            off = off + jnp.where(t < s, p_t, 0)
            total = total + p_t
            cnt_s = cnt_s + jnp.where(t == s, v_t, 0)

        stage_v[...] = jnp.full((16,), total, jnp.int32)

        @pl.when(s == 0)
        def _():
            pltpu.sync_copy(stage_v, cnt_hbm.at[pl.ds(c * 16, 16)])

        for k in range(nvec):
            idx_v[pl.ds(k * 16, 16)] = jnp.zeros((16,), jnp.int32)
        carry = jnp.int32(0)
        for k in range(nvec):
            v = mask_v[pl.ds(s * chunk + k * 16, 16)]
            bm = v > 0
            cum = plsc.cumsum(v)
            lpos = cum - 1 + carry
            iv = row0 + base + k * 16 + lax.iota(jnp.int32, 16)
            zsrc = c * reg + half + jnp.bitwise_and(iv, 255)
            srcv = jnp.where(bm, lpos + off + c * reg, zsrc)
            src_v[pl.ds(k * 16, 16)] = srcv
            plsc.store_scatter(idx_v, [lpos], iv, mask=bm)
            carry = carry + jnp.sum(v)

        pltpu.sync_copy(src_v, src_hbm.at[pl.ds(base, chunk)])

        dst = c * reg + off
        nchunks = (cnt_s + CH - 1) // CH

        def start_gather(q, b):
            pltpu.async_copy(
                x_hbm.at[idx_v.at[pl.ds(q * CH, CH)]], rbs[b], gsems[b])

        def start_write(q, b):
            pltpu.async_copy(
                rbs[b],
                cx_hbm.at[pl.ds(pl.multiple_of(dst + q * CH, CH), CH)],
                wsems[b])

        for b in range(NBUF):
            @pl.when(b < nchunks)
            def _(b=b):
                start_gather(b, b)

        def ring_step(qo, _):
            for b in range(NBUF):
                q = qo * NBUF + b

                @pl.when(q < nchunks)
                def _(q=q, b=b):
                    pltpu.make_async_copy(
                        x_hbm.at[idx_v.at[pl.ds(q * CH, CH)]], rbs[b],
                        gsems[b]).wait()
                    start_write(q, b)

                    @pl.when(q + NBUF < nchunks)
                    def _(q=q, b=b):
                        pltpu.make_async_copy(
                            rbs[b],
                            cx_hbm.at[pl.ds(
                                pl.multiple_of(dst + q * CH, CH), CH)],
                            wsems[b]).wait()
                        start_gather(q + NBUF, b)
            return 0

        lax.fori_loop(0, (nchunks + NBUF - 1) // NBUF, ring_step, 0)

        for b in range(NBUF):
            @pl.when(b < nchunks)
            def _(b=b):
                pltpu.make_async_copy(
                    rbs[b],
                    cx_hbm.at[pl.ds(pl.multiple_of(dst, CH), CH)],
                    wsems[b]).wait()


    def mm_body(cnt_ref, x_ref, w_ref, b_ref, o_ref):
        i = pl.program_id(0)
        c = (i >= regblk).astype(jnp.int32)
        li = i - c * regblk
        cnt = jnp.where(c == 0, cnt_ref[0], cnt_ref[16])
        nact = (cnt + BMM - 1) // BMM

        @pl.when(li < nact)
        def _():
            acc = jnp.dot(x_ref[...], w_ref[...],
                          preferred_element_type=jnp.float32)
            o_ref[...] = acc + b_ref[...]

        @pl.when(li >= nact)
        def _():
            o_ref[...] = jnp.zeros_like(o_ref)

    def mm_xmap(i, cnt_ref):
        c = (i >= regblk).astype(jnp.int32)
        li = i - c * regblk
        cnt = jnp.where(c == 0, cnt_ref[0], cnt_ref[16])
        nact = (cnt + BMM - 1) // BMM
        return (c * regblk + jnp.minimum(li, jnp.maximum(nact - 1, 0)), 0)

    def mm_omap(i, cnt_ref):
        c = (i >= regblk).astype(jnp.int32)
        li = i - c * regblk
        cnt = jnp.where(c == 0, cnt_ref[0], cnt_ref[16])
        nact = (cnt + BMM - 1) // BMM
        return (c * regblk + jnp.where(li < nact, li, regblk - 1), 0)

    def matmul(cnt, cx, W, b2):
        return pl.pallas_call(
            mm_body,
            grid_spec=pltpu.PrefetchScalarGridSpec(
                num_scalar_prefetch=1,
                grid=(mc // BMM,),
                in_specs=[
                    pl.BlockSpec((BMM, H), mm_xmap),
                    pl.BlockSpec((H, D), lambda i, cnt_ref: (0, 0)),
                    pl.BlockSpec((1, D), lambda i, cnt_ref: (0, 0)),
                ],
                out_specs=pl.BlockSpec((BMM, D), mm_omap),
            ),
            out_shape=jax.ShapeDtypeStruct((mc, D), jnp.float32),
            compiler_params=pltpu.CompilerParams(
                dimension_semantics=("arbitrary",),
            ),
        )(cnt, cx, W, b2)


    @functools.partial(
        pl.kernel,
        out_type=jax.ShapeDtypeStruct((nrows, D), jnp.float32),
        mesh=_mesh,
        scratch_types=[
            pltpu.VMEM((chunk,), jnp.int32),
            pltpu.VMEM((CH2, D), jnp.float32),
            pltpu.VMEM((CH2, D), jnp.float32),
            pltpu.VMEM((CH2, D), jnp.float32),
            pltpu.SemaphoreType.DMA,
            pltpu.SemaphoreType.DMA,
            pltpu.SemaphoreType.DMA,
            pltpu.SemaphoreType.DMA,
            pltpu.SemaphoreType.DMA,
            pltpu.SemaphoreType.DMA,
        ],
        compiler_params=pltpu.CompilerParams(needs_layout_passes=False),
    )
    def sc_expand(co_hbm, src_hbm, out_hbm, src_v, rb0, rb1, rb2,
                  g0, g1, g2, w0, w1, w2):
        rbs = (rb0, rb1, rb2)
        gsems = (g0, g1, g2)
        wsems = (w0, w1, w2)
        c = lax.axis_index("c")
        s = lax.axis_index("s")
        base = c * half + s * chunk
        nchunks = chunk // CH2

        pltpu.sync_copy(src_hbm.at[pl.ds(base, chunk)], src_v)

        def start_gather(q, b):
            pltpu.async_copy(
                co_hbm.at[src_v.at[pl.ds(q * CH2, CH2)]], rbs[b], gsems[b])

        def wait_gather(q, b):
            pltpu.make_async_copy(
                co_hbm.at[src_v.at[pl.ds(q * CH2, CH2)]], rbs[b],
                gsems[b]).wait()

        def start_write(q, b):
            pltpu.async_copy(
                rbs[b], out_hbm.at[pl.ds(base + q * CH2, CH2)], wsems[b])

        def wait_write(q, b):
            pltpu.make_async_copy(
                rbs[b], out_hbm.at[pl.ds(base + q * CH2, CH2)],
                wsems[b]).wait()

        for b in range(NBUF):
            start_gather(b, b)
        for q in range(nchunks):
            b = q % NBUF
            wait_gather(q, b)
            start_write(q, b)
            if q + NBUF < nchunks:
                wait_write(q, b)
                start_gather(q + NBUF, b)
        for q in range(nchunks - NBUF, nchunks):
            wait_write(q, q % NBUF)

    def run(mi, x, W, b2):
        cx, src, cnt = sc_compact(mi, x)
        co = matmul(cnt, cx, W, b2)
        return sc_expand(co, src)

    return run


DENSE_ROWS = 4096


def _dense_body(x_ref, m_ref, w_ref, b_ref, o_ref):
    m = m_ref[...]
    acc = jnp.dot(x_ref[...] * m, w_ref[...],
                  preferred_element_type=jnp.float32)
    o_ref[...] = acc + m * b_ref[...]


def _dense_matmul(x, mf, W, b2):
    return pl.pallas_call(
        _dense_body,
        grid=(DENSE_ROWS // BMM,),
        in_specs=[
            pl.BlockSpec((BMM, H), lambda i: (i, 0)),
            pl.BlockSpec((BMM, 1), lambda i: (i, 0)),
            pl.BlockSpec((H, D), lambda i: (0, 0)),
            pl.BlockSpec((1, D), lambda i: (0, 0)),
        ],
        out_specs=pl.BlockSpec((BMM, D), lambda i: (i, 0)),
        out_shape=jax.ShapeDtypeStruct((DENSE_ROWS, D), jnp.float32),
        compiler_params=pltpu.CompilerParams(
            dimension_semantics=("arbitrary",),
        ),
    )(x, mf, W, b2)


_sparse_run = _make_sparse_stages(DENSE_ROWS, M - DENSE_ROWS)


def kernel(inputs, mask, W, b):
    B, T, _ = inputs.shape
    x = inputs.reshape(M, H)
    mi = mask.reshape(M).astype(jnp.int32)
    b2 = b.reshape(1, D)

    mf_dense = mi[:DENSE_ROWS].reshape(DENSE_ROWS, 1).astype(jnp.float32)
    out_dense = _dense_matmul(x, mf_dense, W, b2)
    out_sparse = _sparse_run(mi, x, W, b2)
    out = jnp.concatenate([out_dense, out_sparse], axis=0)
    return out.reshape(B, T, D)


# speedup vs baseline: 1.2072x; 1.2072x over previous
import functools

import jax
import jax.numpy as jnp
from jax import lax
from jax.experimental import pallas as pl
from jax.experimental.pallas import tpu as pltpu
from jax.experimental.pallas import tpu_sc as plsc

M = 16384
H = 2048
D = 4096
NCORE = 2
NSUB = 16
BMM = 256
CH = 16
CH2 = 8
NBUF = 3

_mesh = plsc.VectorSubcoreMesh(core_axis_name="c", subcore_axis_name="s")


def _make_sparse_stages(row0, nrows):
    half = nrows // NCORE
    chunk = half // NSUB
    nvec = chunk // 16
    regblk = half // BMM + 2
    reg = regblk * BMM
    mc = NCORE * reg
    zpad = 2 * BMM

    @functools.partial(
        pl.kernel,
        out_type=(
            jax.ShapeDtypeStruct((mc, H), jnp.float32),
            jax.ShapeDtypeStruct((nrows,), jnp.int32),
            jax.ShapeDtypeStruct((NCORE * 16,), jnp.int32),
        ),
        mesh=_mesh,
        scratch_types=[
            pltpu.VMEM((half,), jnp.int32),
            pltpu.VMEM((chunk,), jnp.int32),
            pltpu.VMEM((chunk,), jnp.int32),
            pltpu.VMEM((16,), jnp.int32),
            pltpu.VMEM((CH, H), jnp.float32),
            pltpu.VMEM((CH, H), jnp.float32),
            pltpu.VMEM((CH, H), jnp.float32),
            pltpu.SemaphoreType.DMA,
            pltpu.SemaphoreType.DMA,
            pltpu.SemaphoreType.DMA,
            pltpu.SemaphoreType.DMA,
            pltpu.SemaphoreType.DMA,
            pltpu.SemaphoreType.DMA,
        ],
        compiler_params=pltpu.CompilerParams(needs_layout_passes=False),
    )
    def sc_compact(mask_hbm, x_hbm, cx_hbm, src_hbm, cnt_hbm,
                   mask_v, idx_v, src_v, stage_v, rb0, rb1, rb2,
                   g0, g1, g2, w0, w1, w2):
        rbs = (rb0, rb1, rb2)
        gsems = (g0, g1, g2)
        wsems = (w0, w1, w2)
        c = lax.axis_index("c")
        s = lax.axis_index("s")
        base = c * half + s * chunk

        pltpu.sync_copy(mask_hbm.at[pl.ds(row0 + c * half, half)], mask_v)

        off = jnp.int32(0)
        total = jnp.int32(0)
        cnt_s = jnp.int32(0)
        for t in range(NSUB):
            def _vsum(k, a, t=t):
                return a + mask_v[pl.ds(t * chunk + k * 16, 16)]
            v_t = jnp.sum(lax.fori_loop(0, nvec, _vsum,
                                        jnp.zeros((16,), jnp.int32)))
            p_t = ((v_t + CH - 1) // CH) * CH
            off = off + jnp.where(t < s, p_t, 0)
            total = total + p_t
            cnt_s = cnt_s + jnp.where(t == s, v_t, 0)

        stage_v[...] = jnp.full((16,), total, jnp.int32)

        @pl.when(s == 0)
        def _():
            pltpu.sync_copy(stage_v, cnt_hbm.at[pl.ds(c * 16, 16)])

        for k in range(nvec):
            idx_v[pl.ds(k * 16, 16)] = jnp.zeros((16,), jnp.int32)
        carry = jnp.int32(0)
        for k in range(nvec):
            v = mask_v[pl.ds(s * chunk + k * 16, 16)]
            bm = v > 0
            cum = plsc.cumsum(v)
            lpos = cum - 1 + carry
            iv = row0 + base + k * 16 + lax.iota(jnp.int32, 16)
            zsrc = c * reg + half + jnp.bitwise_and(iv, zpad - 1)
            srcv = jnp.where(bm, lpos + off + c * reg, zsrc)
            src_v[pl.ds(k * 16, 16)] = srcv
            plsc.store_scatter(idx_v, [lpos], iv, mask=bm)
            carry = carry + jnp.sum(v)

        pltpu.sync_copy(src_v, src_hbm.at[pl.ds(base, chunk)])

        dst = c * reg + off
        nchunks = (cnt_s + CH - 1) // CH

        def start_gather(q, b):
            pltpu.async_copy(
                x_hbm.at[idx_v.at[pl.ds(q * CH, CH)]], rbs[b], gsems[b])

        def start_write(q, b):
            pltpu.async_copy(
                rbs[b],
                cx_hbm.at[pl.ds(pl.multiple_of(dst + q * CH, CH), CH)],
                wsems[b])

        for b in range(NBUF):
            @pl.when(b < nchunks)
            def _(b=b):
                start_gather(b, b)

        def ring_step(qo, _):
            for b in range(NBUF):
                q = qo * NBUF + b

                @pl.when(q < nchunks)
                def _(q=q, b=b):
                    pltpu.make_async_copy(
                        x_hbm.at[idx_v.at[pl.ds(q * CH, CH)]], rbs[b],
                        gsems[b]).wait()
                    start_write(q, b)

                    @pl.when(q + NBUF < nchunks)
                    def _(q=q, b=b):
                        pltpu.make_async_copy(
                            rbs[b],
                            cx_hbm.at[pl.ds(
                                pl.multiple_of(dst + q * CH, CH), CH)],
                            wsems[b]).wait()
                        start_gather(q + NBUF, b)
            return 0

        lax.fori_loop(0, (nchunks + NBUF - 1) // NBUF, ring_step, 0)

        for b in range(NBUF):
            @pl.when(b < nchunks)
            def _(b=b):
                pltpu.make_async_copy(
                    rbs[b],
                    cx_hbm.at[pl.ds(pl.multiple_of(dst, CH), CH)],
                    wsems[b]).wait()


    def mm_body(cnt_ref, x_ref, w_ref, b_ref, o_ref):
        i = pl.program_id(0)
        c = (i >= regblk).astype(jnp.int32)
        li = i - c * regblk
        cnt = jnp.where(c == 0, cnt_ref[0], cnt_ref[16])
        nact = (cnt + BMM - 1) // BMM

        @pl.when(li < nact)
        def _():
            acc = jnp.dot(x_ref[...], w_ref[...],
                          preferred_element_type=jnp.float32)
            o_ref[...] = acc + b_ref[...]

        @pl.when(li >= nact)
        def _():
            o_ref[...] = jnp.zeros_like(o_ref)

    def mm_xmap(i, cnt_ref):
        c = (i >= regblk).astype(jnp.int32)
        li = i - c * regblk
        cnt = jnp.where(c == 0, cnt_ref[0], cnt_ref[16])
        nact = (cnt + BMM - 1) // BMM
        return (c * regblk + jnp.minimum(li, jnp.maximum(nact - 1, 0)), 0)

    def mm_omap(i, cnt_ref):
        c = (i >= regblk).astype(jnp.int32)
        li = i - c * regblk
        cnt = jnp.where(c == 0, cnt_ref[0], cnt_ref[16])
        nact = (cnt + BMM - 1) // BMM
        mid = (nact + regblk) // 2
        zblk = jnp.where(li < mid, regblk - 2, regblk - 1)
        return (c * regblk + jnp.where(li < nact, li, zblk), 0)

    def matmul(cnt, cx, W, b2):
        return pl.pallas_call(
            mm_body,
            grid_spec=pltpu.PrefetchScalarGridSpec(
                num_scalar_prefetch=1,
                grid=(mc // BMM,),
                in_specs=[
                    pl.BlockSpec((BMM, H), mm_xmap),
                    pl.BlockSpec((H, D), lambda i, cnt_ref: (0, 0)),
                    pl.BlockSpec((1, D), lambda i, cnt_ref: (0, 0)),
                ],
                out_specs=pl.BlockSpec((BMM, D), mm_omap),
            ),
            out_shape=jax.ShapeDtypeStruct((mc, D), jnp.float32),
            compiler_params=pltpu.CompilerParams(
                dimension_semantics=("arbitrary",),
            ),
        )(cnt, cx, W, b2)


    @functools.partial(
        pl.kernel,
        out_type=jax.ShapeDtypeStruct((nrows, D), jnp.float32),
        mesh=_mesh,
        scratch_types=[
            pltpu.VMEM((chunk,), jnp.int32),
            pltpu.VMEM((CH2, D), jnp.float32),
            pltpu.VMEM((CH2, D), jnp.float32),
            pltpu.VMEM((CH2, D), jnp.float32),
            pltpu.SemaphoreType.DMA,
            pltpu.SemaphoreType.DMA,
            pltpu.SemaphoreType.DMA,
            pltpu.SemaphoreType.DMA,
            pltpu.SemaphoreType.DMA,
            pltpu.SemaphoreType.DMA,
        ],
        compiler_params=pltpu.CompilerParams(needs_layout_passes=False),
    )
    def sc_expand(co_hbm, src_hbm, out_hbm, src_v, rb0, rb1, rb2,
                  g0, g1, g2, w0, w1, w2):
        rbs = (rb0, rb1, rb2)
        gsems = (g0, g1, g2)
        wsems = (w0, w1, w2)
        c = lax.axis_index("c")
        s = lax.axis_index("s")
        base = c * half + s * chunk
        nchunks = chunk // CH2

        pltpu.sync_copy(src_hbm.at[pl.ds(base, chunk)], src_v)

        def start_gather(q, b):
            pltpu.async_copy(
                co_hbm.at[src_v.at[pl.ds(q * CH2, CH2)]], rbs[b], gsems[b])

        def wait_gather(q, b):
            pltpu.make_async_copy(
                co_hbm.at[src_v.at[pl.ds(q * CH2, CH2)]], rbs[b],
                gsems[b]).wait()

        def start_write(q, b):
            pltpu.async_copy(
                rbs[b], out_hbm.at[pl.ds(base + q * CH2, CH2)], wsems[b])

        def wait_write(q, b):
            pltpu.make_async_copy(
                rbs[b], out_hbm.at[pl.ds(base + q * CH2, CH2)],
                wsems[b]).wait()

        for b in range(NBUF):
            start_gather(b, b)
        for q in range(nchunks):
            b = q % NBUF
            wait_gather(q, b)
            start_write(q, b)
            if q + NBUF < nchunks:
                wait_write(q, b)
                start_gather(q + NBUF, b)
        for q in range(nchunks - NBUF, nchunks):
            wait_write(q, q % NBUF)

    def run(mi, x, W, b2):
        cx, src, cnt = sc_compact(mi, x)
        co = matmul(cnt, cx, W, b2)
        return sc_expand(co, src)

    return run


_sparse_run = _make_sparse_stages(0, M)


def kernel(inputs, mask, W, b):
    B, T, _ = inputs.shape
    x = inputs.reshape(M, H)
    mi = mask.reshape(M).astype(jnp.int32)
    b2 = b.reshape(1, D)
    out = _sparse_run(mi, x, W, b2)
    return out.reshape(B, T, D)


# speedup vs baseline: 1.2216x; 1.0119x over previous
import functools

import jax
import jax.numpy as jnp
from jax import lax
from jax.experimental import pallas as pl
from jax.experimental.pallas import tpu as pltpu
from jax.experimental.pallas import tpu_sc as plsc

M = 16384
H = 2048
D = 4096
NCORE = 2
NSUB = 16
BMM = 256
CH = 16
CH2 = 8
NBUF = 3

_mesh = plsc.VectorSubcoreMesh(core_axis_name="c", subcore_axis_name="s")


def _make_sparse_stages(row0, nrows):
    half = nrows // NCORE
    chunk = half // NSUB
    nvec = chunk // 16
    regblk = half // BMM + 1
    reg = regblk * BMM
    mc = NCORE * reg
    zpad = BMM

    @functools.partial(
        pl.kernel,
        out_type=(
            jax.ShapeDtypeStruct((mc, H), jnp.float32),
            jax.ShapeDtypeStruct((nrows,), jnp.int32),
            jax.ShapeDtypeStruct((NCORE * 16,), jnp.int32),
        ),
        mesh=_mesh,
        scratch_types=[
            pltpu.VMEM((half,), jnp.int32),
            pltpu.VMEM((chunk,), jnp.int32),
            pltpu.VMEM((chunk,), jnp.int32),
            pltpu.VMEM((16,), jnp.int32),
            pltpu.VMEM((CH, H), jnp.float32),
            pltpu.VMEM((CH, H), jnp.float32),
            pltpu.VMEM((CH, H), jnp.float32),
            pltpu.SemaphoreType.DMA,
            pltpu.SemaphoreType.DMA,
            pltpu.SemaphoreType.DMA,
            pltpu.SemaphoreType.DMA,
            pltpu.SemaphoreType.DMA,
            pltpu.SemaphoreType.DMA,
        ],
        compiler_params=pltpu.CompilerParams(needs_layout_passes=False),
    )
    def sc_compact(mask_hbm, x_hbm, cx_hbm, src_hbm, cnt_hbm,
                   mask_v, idx_v, src_v, stage_v, rb0, rb1, rb2,
                   g0, g1, g2, w0, w1, w2):
        rbs = (rb0, rb1, rb2)
        gsems = (g0, g1, g2)
        wsems = (w0, w1, w2)
        c = lax.axis_index("c")
        s = lax.axis_index("s")
        base = c * half + s * chunk

        pltpu.sync_copy(mask_hbm.at[pl.ds(row0 + c * half, half)], mask_v)

        off = jnp.int32(0)
        total = jnp.int32(0)
        cnt_s = jnp.int32(0)
        for t in range(NSUB):
            def _vsum(k, a, t=t):
                return a + mask_v[pl.ds(t * chunk + k * 16, 16)]
            v_t = jnp.sum(lax.fori_loop(0, nvec, _vsum,
                                        jnp.zeros((16,), jnp.int32)))
            p_t = ((v_t + CH - 1) // CH) * CH
            off = off + jnp.where(t < s, p_t, 0)
            total = total + p_t
            cnt_s = cnt_s + jnp.where(t == s, v_t, 0)

        stage_v[...] = jnp.full((16,), total, jnp.int32)

        @pl.when(s == 0)
        def _():
            pltpu.sync_copy(stage_v, cnt_hbm.at[pl.ds(c * 16, 16)])

        for k in range(nvec):
            idx_v[pl.ds(k * 16, 16)] = jnp.zeros((16,), jnp.int32)
        carry = jnp.int32(0)
        for k in range(nvec):
            v = mask_v[pl.ds(s * chunk + k * 16, 16)]
            bm = v > 0
            cum = plsc.cumsum(v)
            lpos = cum - 1 + carry
            iv = row0 + base + k * 16 + lax.iota(jnp.int32, 16)
            zsrc = c * reg + half + jnp.bitwise_and(iv, zpad - 1)
            srcv = jnp.where(bm, lpos + off + c * reg, zsrc)
            src_v[pl.ds(k * 16, 16)] = srcv
            plsc.store_scatter(idx_v, [lpos], iv, mask=bm)
            carry = carry + jnp.sum(v)

        pltpu.sync_copy(src_v, src_hbm.at[pl.ds(base, chunk)])

        dst = c * reg + off
        nchunks = (cnt_s + CH - 1) // CH

        def start_gather(q, b):
            pltpu.async_copy(
                x_hbm.at[idx_v.at[pl.ds(q * CH, CH)]], rbs[b], gsems[b])

        def start_write(q, b):
            pltpu.async_copy(
                rbs[b],
                cx_hbm.at[pl.ds(pl.multiple_of(dst + q * CH, CH), CH)],
                wsems[b])

        for b in range(NBUF):
            @pl.when(b < nchunks)
            def _(b=b):
                start_gather(b, b)

        def ring_step(qo, _):
            for b in range(NBUF):
                q = qo * NBUF + b

                @pl.when(q < nchunks)
                def _(q=q, b=b):
                    pltpu.make_async_copy(
                        x_hbm.at[idx_v.at[pl.ds(q * CH, CH)]], rbs[b],
                        gsems[b]).wait()
                    start_write(q, b)

                    @pl.when(q + NBUF < nchunks)
                    def _(q=q, b=b):
                        pltpu.make_async_copy(
                            rbs[b],
                            cx_hbm.at[pl.ds(
                                pl.multiple_of(dst + q * CH, CH), CH)],
                            wsems[b]).wait()
                        start_gather(q + NBUF, b)
            return 0

        lax.fori_loop(0, (nchunks + NBUF - 1) // NBUF, ring_step, 0)

        for b in range(NBUF):
            @pl.when(b < nchunks)
            def _(b=b):
                pltpu.make_async_copy(
                    rbs[b],
                    cx_hbm.at[pl.ds(pl.multiple_of(dst, CH), CH)],
                    wsems[b]).wait()


    def mm_body(cnt_ref, x_ref, w_ref, b_ref, o_ref):
        i = pl.program_id(0)
        c = (i >= regblk).astype(jnp.int32)
        li = i - c * regblk
        cnt = jnp.where(c == 0, cnt_ref[0], cnt_ref[16])
        nact = (cnt + BMM - 1) // BMM

        @pl.when(li < nact)
        def _():
            acc = jnp.dot(x_ref[...], w_ref[...],
                          preferred_element_type=jnp.float32)
            o_ref[...] = acc + b_ref[...]

        @pl.when(li >= nact)
        def _():
            o_ref[...] = jnp.zeros_like(o_ref)

    def mm_xmap(i, cnt_ref):
        c = (i >= regblk).astype(jnp.int32)
        li = i - c * regblk
        cnt = jnp.where(c == 0, cnt_ref[0], cnt_ref[16])
        nact = (cnt + BMM - 1) // BMM
        return (c * regblk + jnp.minimum(li, jnp.maximum(nact - 1, 0)), 0)

    def mm_omap(i, cnt_ref):
        c = (i >= regblk).astype(jnp.int32)
        li = i - c * regblk
        cnt = jnp.where(c == 0, cnt_ref[0], cnt_ref[16])
        nact = (cnt + BMM - 1) // BMM
        return (c * regblk + jnp.where(li < nact, li, regblk - 1), 0)

    def matmul(cnt, cx, W, b2):
        return pl.pallas_call(
            mm_body,
            grid_spec=pltpu.PrefetchScalarGridSpec(
                num_scalar_prefetch=1,
                grid=(mc // BMM,),
                in_specs=[
                    pl.BlockSpec((BMM, H), mm_xmap),
                    pl.BlockSpec((H, D), lambda i, cnt_ref: (0, 0)),
                    pl.BlockSpec((1, D), lambda i, cnt_ref: (0, 0)),
                ],
                out_specs=pl.BlockSpec((BMM, D), mm_omap),
            ),
            out_shape=jax.ShapeDtypeStruct((mc, D), jnp.float32),
            compiler_params=pltpu.CompilerParams(
                dimension_semantics=("arbitrary",),
            ),
        )(cnt, cx, W, b2)


    @functools.partial(
        pl.kernel,
        out_type=jax.ShapeDtypeStruct((nrows, D), jnp.float32),
        mesh=_mesh,
        scratch_types=[
            pltpu.VMEM((chunk,), jnp.int32),
            pltpu.VMEM((CH2, D), jnp.float32),
            pltpu.VMEM((CH2, D), jnp.float32),
            pltpu.VMEM((CH2, D), jnp.float32),
            pltpu.SemaphoreType.DMA,
            pltpu.SemaphoreType.DMA,
            pltpu.SemaphoreType.DMA,
            pltpu.SemaphoreType.DMA,
            pltpu.SemaphoreType.DMA,
            pltpu.SemaphoreType.DMA,
        ],
        compiler_params=pltpu.CompilerParams(needs_layout_passes=False),
    )
    def sc_expand(co_hbm, src_hbm, out_hbm, src_v, rb0, rb1, rb2,
                  g0, g1, g2, w0, w1, w2):
        rbs = (rb0, rb1, rb2)
        gsems = (g0, g1, g2)
        wsems = (w0, w1, w2)
        c = lax.axis_index("c")
        s = lax.axis_index("s")
        base = c * half + s * chunk
        nchunks = chunk // CH2

        pltpu.sync_copy(src_hbm.at[pl.ds(base, chunk)], src_v)

        def start_gather(q, b):
            pltpu.async_copy(
                co_hbm.at[src_v.at[pl.ds(q * CH2, CH2)]], rbs[b], gsems[b])

        def wait_gather(q, b):
            pltpu.make_async_copy(
                co_hbm.at[src_v.at[pl.ds(q * CH2, CH2)]], rbs[b],
                gsems[b]).wait()

        def start_write(q, b):
            pltpu.async_copy(
                rbs[b], out_hbm.at[pl.ds(base + q * CH2, CH2)], wsems[b])

        def wait_write(q, b):
            pltpu.make_async_copy(
                rbs[b], out_hbm.at[pl.ds(base + q * CH2, CH2)],
                wsems[b]).wait()

        for b in range(NBUF):
            start_gather(b, b)
        for q in range(nchunks):
            b = q % NBUF
            wait_gather(q, b)
            start_write(q, b)
            if q + NBUF < nchunks:
                wait_write(q, b)
                start_gather(q + NBUF, b)
        for q in range(nchunks - NBUF, nchunks):
            wait_write(q, q % NBUF)

    def run(mi, x, W, b2):
        cx, src, cnt = sc_compact(mi, x)
        co = matmul(cnt, cx, W, b2)
        return sc_expand(co, src)

    return run


_sparse_run = _make_sparse_stages(0, M)


def kernel(inputs, mask, W, b):
    B, T, _ = inputs.shape
    x = inputs.reshape(M, H)
    mi = mask.reshape(M).astype(jnp.int32)
    b2 = b.reshape(1, D)
    out = _sparse_run(mi, x, W, b2)
    return out.reshape(B, T, D)
